# Initial kernel scaffold; baseline (speedup 1.0000x reference)
import jax
import jax.numpy as jnp
from jax.experimental import pallas as pl


def kernel(x, edge_index, W1, b1, W2, b2, W3, b3):
    raise NotImplementedError("write your pallas kernel here")


# speedup vs baseline: 3.4737x; 3.4737x over previous
import jax
import jax.numpy as jnp
from jax import lax
from jax.experimental import pallas as pl
from jax.experimental.pallas import tpu as pltpu
from jax.experimental.pallas import tpu_sc as plsc

N = 10000
D = 128
E = 320000
NP = 10240
DUMP = 10000
NW = 32
CH = 160
B = 64
RPT = NP // 16
BLK = 1024

_SC_MESH = plsc.VectorSubcoreMesh(
    core_axis_name="c", subcore_axis_name="s", num_cores=2, num_subcores=16
)


DEG_Q = 4


def _deg_body(srcb, dstb, ones_hbm, zeros_hbm, out, idx_v, ones_v, hist_sh):
    cid = lax.axis_index("c")
    sid = lax.axis_index("s")
    wid = cid * 16 + sid
    pltpu.sync_copy(ones_hbm, ones_v)
    r0 = sid * RPT
    for which, blk in ((0, srcb), (1, dstb)):
        pltpu.sync_copy(zeros_hbm, hist_sh.at[pl.ds(r0, RPT)])
        plsc.subcore_barrier()
        for q in range(DEG_Q):
            pltpu.sync_copy(blk.at[wid, pl.ds(q * (CH // DEG_Q), CH // DEG_Q)],
                            idx_v)

            def step(g, carry):
                pltpu.sync_copy(ones_v, hist_sh.at[idx_v.at[g]], add=True)
                return carry

            lax.fori_loop(0, CH // DEG_Q, step, 0)
        plsc.subcore_barrier()
        pltpu.sync_copy(hist_sh.at[pl.ds(r0, RPT)],
                        out.at[cid, which, pl.ds(r0, RPT)])
        plsc.subcore_barrier()


_DEG_FN = pl.kernel(
    _deg_body,
    out_type=jax.ShapeDtypeStruct((2, 2, NP, D), jnp.float32),
    mesh=_SC_MESH,
    scratch_types=[
        pltpu.VMEM((CH // DEG_Q, B), jnp.int32),
        pltpu.VMEM((B, D), jnp.float32),
        pltpu.VMEM_SHARED((NP, D), jnp.float32),
    ],
)


HN = CH // 2


def _gs_body(h_hbm, srcb, dstb, zeros_hbm, out,
             src_v, dst_v, rows0, rows1, agg_sh, sem0, sem1):
    cid = lax.axis_index("c")
    sid = lax.axis_index("s")
    wid = cid * 16 + sid
    r0 = sid * RPT

    def load_half(half):
        pltpu.sync_copy(srcb.at[wid, pl.ds(half * HN, HN)], src_v)
        pltpu.sync_copy(dstb.at[wid, pl.ds(half * HN, HN)], dst_v)
        pltpu.async_copy(h_hbm.at[src_v.at[0]], rows0, sem0)
        pltpu.async_copy(h_hbm.at[src_v.at[1]], rows1, sem1)

    def run_half():
        def step(i, carry):
            g = 2 * i
            pltpu.make_async_copy(h_hbm.at[src_v.at[g]], rows0, sem0).wait()
            pltpu.sync_copy(rows0, agg_sh.at[dst_v.at[g]], add=True)

            @pl.when(g + 2 < HN)
            def _():
                pltpu.async_copy(h_hbm.at[src_v.at[g + 2]], rows0, sem0)

            pltpu.make_async_copy(h_hbm.at[src_v.at[g + 1]], rows1, sem1).wait()
            pltpu.sync_copy(rows1, agg_sh.at[dst_v.at[g + 1]], add=True)

            @pl.when(g + 3 < HN)
            def _():
                pltpu.async_copy(h_hbm.at[src_v.at[g + 3]], rows1, sem1)

            return carry

        lax.fori_loop(0, HN // 2, step, 0)

    load_half(0)
    pltpu.sync_copy(zeros_hbm, agg_sh.at[pl.ds(r0, RPT)])
    plsc.subcore_barrier()
    run_half()
    load_half(1)
    run_half()
    plsc.subcore_barrier()
    pltpu.sync_copy(agg_sh.at[pl.ds(r0, RPT)], out.at[cid, pl.ds(r0, RPT)])


_GS_FN = pl.kernel(
    _gs_body,
    out_type=jax.ShapeDtypeStruct((2, NP, D), jnp.float32),
    mesh=_SC_MESH,
    scratch_types=[
        pltpu.VMEM((HN, B), jnp.int32),
        pltpu.VMEM((HN, B), jnp.int32),
        pltpu.VMEM((B, D), jnp.float32),
        pltpu.VMEM((B, D), jnp.float32),
        pltpu.VMEM_SHARED((NP, D), jnp.float32),
        pltpu.SemaphoreType.DMA,
        pltpu.SemaphoreType.DMA,
    ],
)


def _pre_body(dp_ref, x_ref, w_ref, ns_ref, nd_ref, h_ref):
    dp = dp_ref[...]
    dsrc = dp[0, :, 0:1] + dp[2, :, 0:1]
    ddst = dp[1, :, 0:1] + dp[3, :, 0:1]
    ns = jnp.where(dsrc > 0, lax.rsqrt(jnp.maximum(dsrc, 1.0)), 0.0)
    nd = jnp.where(ddst > 0, lax.rsqrt(jnp.maximum(ddst, 1.0)), 0.0)
    ns_bc = jnp.broadcast_to(ns, (BLK, D))
    nd_bc = jnp.broadcast_to(nd, (BLK, D))
    ns_ref[...] = ns_bc
    nd_ref[...] = nd_bc
    h_ref[...] = jnp.dot(x_ref[...] * ns_bc, w_ref[...],
                         preferred_element_type=jnp.float32)


def _pre_call(dp, xp, W1):
    return pl.pallas_call(
        _pre_body,
        grid=(NP // BLK,),
        in_specs=[
            pl.BlockSpec((4, BLK, D), lambda i: (0, i, 0)),
            pl.BlockSpec((BLK, D), lambda i: (i, 0)),
            pl.BlockSpec((D, D), lambda i: (0, 0)),
        ],
        out_specs=[pl.BlockSpec((BLK, D), lambda i: (i, 0))] * 3,
        out_shape=[jax.ShapeDtypeStruct((NP, D), jnp.float32)] * 3,
    )(dp, xp, W1)


def _mid_body(p_ref, nd_ref, ns_ref, b_ref, w_ref, h_ref):
    t = (p_ref[0] + p_ref[1]) * nd_ref[...] + b_ref[...]
    t = jnp.maximum(t, 0.0)
    h_ref[...] = jnp.dot(t * ns_ref[...], w_ref[...],
                         preferred_element_type=jnp.float32)


def _mid_call(p, nd, ns, b, Wn):
    return pl.pallas_call(
        _mid_body,
        grid=(NP // BLK,),
        in_specs=[
            pl.BlockSpec((2, BLK, D), lambda i: (0, i, 0)),
            pl.BlockSpec((BLK, D), lambda i: (i, 0)),
            pl.BlockSpec((BLK, D), lambda i: (i, 0)),
            pl.BlockSpec((1, D), lambda i: (0, 0)),
            pl.BlockSpec((D, D), lambda i: (0, 0)),
        ],
        out_specs=pl.BlockSpec((BLK, D), lambda i: (i, 0)),
        out_shape=jax.ShapeDtypeStruct((NP, D), jnp.float32),
    )(p, nd, ns, b, Wn)


def _fin_body(p_ref, nd_ref, b_ref, o_ref):
    o_ref[...] = (p_ref[0] + p_ref[1]) * nd_ref[...] + b_ref[...]


def _fin_call(p, nd, b):
    return pl.pallas_call(
        _fin_body,
        grid=(NP // BLK,),
        in_specs=[
            pl.BlockSpec((2, BLK, D), lambda i: (0, i, 0)),
            pl.BlockSpec((BLK, D), lambda i: (i, 0)),
            pl.BlockSpec((1, D), lambda i: (0, 0)),
        ],
        out_specs=pl.BlockSpec((BLK, D), lambda i: (i, 0)),
        out_shape=jax.ShapeDtypeStruct((NP, D), jnp.float32),
    )(p, nd, b)


def kernel(x, edge_index, W1, b1, W2, b2, W3, b3):
    src = edge_index[0]
    dst = edge_index[1]
    pad = NW * CH * B - E
    fill = jnp.full((pad,), DUMP, jnp.int32)
    srcb = jnp.concatenate([src, fill]).reshape(NW, CH, B)
    dstb = jnp.concatenate([dst, fill]).reshape(NW, CH, B)
    xp = jnp.zeros((NP, D), jnp.float32).at[:N].set(x)
    zeros_rows = jnp.zeros((RPT, D), jnp.float32)
    ones_rows = jnp.ones((B, D), jnp.float32)

    degp = _DEG_FN(srcb, dstb, ones_rows, zeros_rows)
    dp = degp.reshape(4, NP, D)
    ns, nd, h = _pre_call(dp, xp, W1)

    p = _GS_FN(h, srcb, dstb, zeros_rows)
    h = _mid_call(p, nd, ns, b1.reshape(1, D), W2)
    p = _GS_FN(h, srcb, dstb, zeros_rows)
    h = _mid_call(p, nd, ns, b2.reshape(1, D), W3)
    p = _GS_FN(h, srcb, dstb, zeros_rows)
    out = _fin_call(p, nd, b3.reshape(1, D))

    return (out[:N], edge_index)


# speedup vs baseline: 9.6750x; 2.7852x over previous
import jax
import jax.numpy as jnp
from jax import lax
from jax.experimental import pallas as pl
from jax.experimental.pallas import tpu as pltpu
from jax.experimental.pallas import tpu_sc as plsc

N = 10000
D = 128
E = 320000
NP = 10240
DUMP = 10000
NW = 32
CH = 80
B = 128
RPT = NP // 16
BLK = 1024

_SC_MESH = plsc.VectorSubcoreMesh(
    core_axis_name="c", subcore_axis_name="s", num_cores=2, num_subcores=16
)


DEG_Q = 5


def _deg_body(srcb, dstb, ones_hbm, zeros_hbm, out, idx_v, ones_v, hist_sh):
    cid = lax.axis_index("c")
    sid = lax.axis_index("s")
    wid = cid * 16 + sid
    pltpu.sync_copy(ones_hbm, ones_v)
    r0 = sid * RPT
    for which, blk in ((0, srcb), (1, dstb)):
        pltpu.sync_copy(zeros_hbm, hist_sh.at[pl.ds(r0, RPT)])
        plsc.subcore_barrier()
        for q in range(DEG_Q):
            pltpu.sync_copy(blk.at[wid, pl.ds(q * (CH // DEG_Q), CH // DEG_Q)],
                            idx_v)

            def step(g, carry):
                pltpu.sync_copy(ones_v, hist_sh.at[idx_v.at[g]], add=True)
                return carry

            lax.fori_loop(0, CH // DEG_Q, step, 0)
        plsc.subcore_barrier()
        pltpu.sync_copy(hist_sh.at[pl.ds(r0, RPT)],
                        out.at[cid, which, pl.ds(r0, RPT)])
        plsc.subcore_barrier()


_DEG_FN = pl.kernel(
    _deg_body,
    out_type=jax.ShapeDtypeStruct((2, 2, NP, D), jnp.float32),
    mesh=_SC_MESH,
    scratch_types=[
        pltpu.VMEM((CH // DEG_Q, B), jnp.int32),
        pltpu.VMEM((B, D), jnp.float32),
        pltpu.VMEM_SHARED((NP, D), jnp.float32),
    ],
)


NSTG = 10
SCH = CH // NSTG


def _gs_body(h_hbm, srcb, dstb, zeros_hbm, out,
             src0, dst0, src1, dst1, rows0, rows1, agg_sh,
             sem0, sem1, semi):
    cid = lax.axis_index("c")
    sid = lax.axis_index("s")
    wid = cid * 16 + sid
    r0 = sid * RPT
    srcs, dsts = (src0, src1), (dst0, dst1)
    rows, sems = (rows0, rows1), (sem0, sem1)

    pltpu.sync_copy(srcb.at[wid, pl.ds(0, SCH)], src0)
    pltpu.sync_copy(dstb.at[wid, pl.ds(0, SCH)], dst0)
    pltpu.async_copy(h_hbm.at[src0.at[0]], rows0, sem0)
    pltpu.async_copy(h_hbm.at[src0.at[1]], rows1, sem1)
    pltpu.sync_copy(zeros_hbm, agg_sh.at[pl.ds(r0, RPT)])
    plsc.subcore_barrier()

    for s in range(NSTG):
        cs, cd = srcs[s % 2], dsts[s % 2]
        nx_s, nx_d = srcs[(s + 1) % 2], dsts[(s + 1) % 2]
        if s + 1 < NSTG:
            pltpu.async_copy(srcb.at[wid, pl.ds((s + 1) * SCH, SCH)],
                             nx_s, semi)
            pltpu.async_copy(dstb.at[wid, pl.ds((s + 1) * SCH, SCH)],
                             nx_d, semi)

        def pair(i, carry, cs=cs, cd=cd):
            g = 2 * i
            for b in range(2):
                j = g + b
                pltpu.make_async_copy(h_hbm.at[cs.at[j]],
                                      rows[b], sems[b]).wait()
                pltpu.sync_copy(rows[b], agg_sh.at[cd.at[j]], add=True)
                pltpu.async_copy(h_hbm.at[cs.at[j + 2]], rows[b], sems[b])
            return carry

        lax.fori_loop(0, SCH // 2 - 1, pair, 0)
        for b in range(2):
            j = SCH - 2 + b
            pltpu.make_async_copy(h_hbm.at[cs.at[j]], rows[b], sems[b]).wait()
            pltpu.sync_copy(rows[b], agg_sh.at[cd.at[j]], add=True)
            if s + 1 < NSTG:
                if b == 0:
                    pltpu.make_async_copy(
                        srcb.at[wid, pl.ds(0, SCH)], nx_s, semi).wait()
                    pltpu.make_async_copy(
                        dstb.at[wid, pl.ds(0, SCH)], nx_d, semi).wait()
                pltpu.async_copy(h_hbm.at[nx_s.at[b]], rows[b], sems[b])

    plsc.subcore_barrier()
    pltpu.sync_copy(agg_sh.at[pl.ds(r0, RPT)], out.at[cid, pl.ds(r0, RPT)])


_GS_FN = pl.kernel(
    _gs_body,
    out_type=jax.ShapeDtypeStruct((2, NP, D), jnp.float32),
    mesh=_SC_MESH,
    scratch_types=[
        pltpu.VMEM((SCH, B), jnp.int32),
        pltpu.VMEM((SCH, B), jnp.int32),
        pltpu.VMEM((SCH, B), jnp.int32),
        pltpu.VMEM((SCH, B), jnp.int32),
        pltpu.VMEM((B, D), jnp.float32),
        pltpu.VMEM((B, D), jnp.float32),
        pltpu.VMEM_SHARED((NP, D), jnp.float32),
        pltpu.SemaphoreType.DMA,
        pltpu.SemaphoreType.DMA,
        pltpu.SemaphoreType.DMA,
    ],
)


def _pre_body(dp_ref, x_ref, w_ref, ns_ref, nd_ref, h_ref):
    dp = dp_ref[...]
    dsrc = dp[0, :, 0:1] + dp[2, :, 0:1]
    ddst = dp[1, :, 0:1] + dp[3, :, 0:1]
    ns = jnp.where(dsrc > 0, lax.rsqrt(jnp.maximum(dsrc, 1.0)), 0.0)
    nd = jnp.where(ddst > 0, lax.rsqrt(jnp.maximum(ddst, 1.0)), 0.0)
    ns_bc = jnp.broadcast_to(ns, (BLK, D))
    nd_bc = jnp.broadcast_to(nd, (BLK, D))
    ns_ref[...] = ns_bc
    nd_ref[...] = nd_bc
    h_ref[...] = jnp.dot(x_ref[...] * ns_bc, w_ref[...],
                         preferred_element_type=jnp.float32)


def _pre_call(dp, xp, W1):
    return pl.pallas_call(
        _pre_body,
        grid=(NP // BLK,),
        in_specs=[
            pl.BlockSpec((4, BLK, D), lambda i: (0, i, 0)),
            pl.BlockSpec((BLK, D), lambda i: (i, 0)),
            pl.BlockSpec((D, D), lambda i: (0, 0)),
        ],
        out_specs=[pl.BlockSpec((BLK, D), lambda i: (i, 0))] * 3,
        out_shape=[jax.ShapeDtypeStruct((NP, D), jnp.float32)] * 3,
    )(dp, xp, W1)


def _mid_body(p_ref, nd_ref, ns_ref, b_ref, w_ref, h_ref):
    t = (p_ref[0] + p_ref[1]) * nd_ref[...] + b_ref[...]
    t = jnp.maximum(t, 0.0)
    h_ref[...] = jnp.dot(t * ns_ref[...], w_ref[...],
                         preferred_element_type=jnp.float32)


def _mid_call(p, nd, ns, b, Wn):
    return pl.pallas_call(
        _mid_body,
        grid=(NP // BLK,),
        in_specs=[
            pl.BlockSpec((2, BLK, D), lambda i: (0, i, 0)),
            pl.BlockSpec((BLK, D), lambda i: (i, 0)),
            pl.BlockSpec((BLK, D), lambda i: (i, 0)),
            pl.BlockSpec((1, D), lambda i: (0, 0)),
            pl.BlockSpec((D, D), lambda i: (0, 0)),
        ],
        out_specs=pl.BlockSpec((BLK, D), lambda i: (i, 0)),
        out_shape=jax.ShapeDtypeStruct((NP, D), jnp.float32),
    )(p, nd, ns, b, Wn)


def _fin_body(p_ref, nd_ref, b_ref, o_ref):
    o_ref[...] = (p_ref[0] + p_ref[1]) * nd_ref[...] + b_ref[...]


def _fin_call(p, nd, b):
    return pl.pallas_call(
        _fin_body,
        grid=(NP // BLK,),
        in_specs=[
            pl.BlockSpec((2, BLK, D), lambda i: (0, i, 0)),
            pl.BlockSpec((BLK, D), lambda i: (i, 0)),
            pl.BlockSpec((1, D), lambda i: (0, 0)),
        ],
        out_specs=pl.BlockSpec((BLK, D), lambda i: (i, 0)),
        out_shape=jax.ShapeDtypeStruct((NP, D), jnp.float32),
    )(p, nd, b)


def kernel(x, edge_index, W1, b1, W2, b2, W3, b3):
    src = edge_index[0]
    dst = edge_index[1]
    pad = NW * CH * B - E
    fill = DUMP + (jnp.arange(pad, dtype=jnp.int32) % (NP - DUMP))
    srcb = jnp.concatenate([src, fill]).reshape(NW, CH, B)
    dstb = jnp.concatenate([dst, fill]).reshape(NW, CH, B)
    xp = jnp.zeros((NP, D), jnp.float32).at[:N].set(x)
    zeros_rows = jnp.zeros((RPT, D), jnp.float32)
    ones_rows = jnp.ones((B, D), jnp.float32)

    degp = _DEG_FN(srcb, dstb, ones_rows, zeros_rows)
    dp = degp.reshape(4, NP, D)
    ns, nd, h = _pre_call(dp, xp, W1)

    p = _GS_FN(h, srcb, dstb, zeros_rows)
    h = _mid_call(p, nd, ns, b1.reshape(1, D), W2)
    p = _GS_FN(h, srcb, dstb, zeros_rows)
    h = _mid_call(p, nd, ns, b2.reshape(1, D), W3)
    p = _GS_FN(h, srcb, dstb, zeros_rows)
    out = _fin_call(p, nd, b3.reshape(1, D))

    return (out[:N], edge_index)


# speedup vs baseline: 9.6983x; 1.0024x over previous
import jax
import jax.numpy as jnp
from jax import lax
from jax.experimental import pallas as pl
from jax.experimental.pallas import tpu as pltpu
from jax.experimental.pallas import tpu_sc as plsc

N = 10000
D = 128
E = 320000
NP = 10240
DUMP = 10000
NW = 32
CH = 80
B = 128
RPT = NP // 16
BLK = 1024

_SC_MESH = plsc.VectorSubcoreMesh(
    core_axis_name="c", subcore_axis_name="s", num_cores=2, num_subcores=16
)


DEG_Q = 5


def _deg_body(srcb, dstb, ones_hbm, zeros_hbm, out, idx_v, ones_v, hist_sh,
              sem_s):
    cid = lax.axis_index("c")
    sid = lax.axis_index("s")
    wid = cid * 16 + sid
    pltpu.sync_copy(ones_hbm, ones_v)
    r0 = sid * RPT
    qch = CH // DEG_Q
    for which, blk in ((0, srcb), (1, dstb)):
        pltpu.sync_copy(zeros_hbm, hist_sh.at[pl.ds(r0, RPT)])
        plsc.subcore_barrier()
        for q in range(DEG_Q):
            pltpu.sync_copy(blk.at[wid, pl.ds(q * qch, qch)], idx_v)

            def fire(g, carry):
                pltpu.async_copy(ones_v, hist_sh.at[idx_v.at[g]], sem_s,
                                 add=True)
                return carry

            def drain(g, carry):
                pltpu.make_async_copy(ones_v, hist_sh.at[idx_v.at[0]],
                                      sem_s).wait()
                return carry

            lax.fori_loop(0, qch, fire, 0)
            lax.fori_loop(0, qch, drain, 0)
        plsc.subcore_barrier()
        pltpu.sync_copy(hist_sh.at[pl.ds(r0, RPT)],
                        out.at[cid, which, pl.ds(r0, RPT)])
        plsc.subcore_barrier()


_DEG_FN = pl.kernel(
    _deg_body,
    out_type=jax.ShapeDtypeStruct((2, 2, NP, D), jnp.float32),
    mesh=_SC_MESH,
    scratch_types=[
        pltpu.VMEM((CH // DEG_Q, B), jnp.int32),
        pltpu.VMEM((B, D), jnp.float32),
        pltpu.VMEM_SHARED((NP, D), jnp.float32),
        pltpu.SemaphoreType.DMA,
    ],
)


NSTG = 10
SCH = CH // NSTG


def _gs_body(h_hbm, srcb, dstb, zeros_hbm, out,
             src0, dst0, src1, dst1, rows0, rows1, agg_sh,
             sem0, sem1, semi):
    cid = lax.axis_index("c")
    sid = lax.axis_index("s")
    wid = cid * 16 + sid
    r0 = sid * RPT
    srcs, dsts = (src0, src1), (dst0, dst1)
    rows, sems = (rows0, rows1), (sem0, sem1)

    pltpu.sync_copy(srcb.at[wid, pl.ds(0, SCH)], src0)
    pltpu.sync_copy(dstb.at[wid, pl.ds(0, SCH)], dst0)
    pltpu.async_copy(h_hbm.at[src0.at[0]], rows0, sem0)
    pltpu.async_copy(h_hbm.at[src0.at[1]], rows1, sem1)
    pltpu.sync_copy(zeros_hbm, agg_sh.at[pl.ds(r0, RPT)])
    plsc.subcore_barrier()

    for s in range(NSTG):
        cs, cd = srcs[s % 2], dsts[s % 2]
        nx_s, nx_d = srcs[(s + 1) % 2], dsts[(s + 1) % 2]
        if s + 1 < NSTG:
            pltpu.async_copy(srcb.at[wid, pl.ds((s + 1) * SCH, SCH)],
                             nx_s, semi)
            pltpu.async_copy(dstb.at[wid, pl.ds((s + 1) * SCH, SCH)],
                             nx_d, semi)

        def pair(i, carry, cs=cs, cd=cd):
            g = 2 * i
            for b in range(2):
                j = g + b
                pltpu.make_async_copy(h_hbm.at[cs.at[j]],
                                      rows[b], sems[b]).wait()
                pltpu.sync_copy(rows[b], agg_sh.at[cd.at[j]], add=True)
                pltpu.async_copy(h_hbm.at[cs.at[j + 2]], rows[b], sems[b])
            return carry

        lax.fori_loop(0, SCH // 2 - 1, pair, 0)
        for b in range(2):
            j = SCH - 2 + b
            pltpu.make_async_copy(h_hbm.at[cs.at[j]], rows[b], sems[b]).wait()
            pltpu.sync_copy(rows[b], agg_sh.at[cd.at[j]], add=True)
            if s + 1 < NSTG:
                if b == 0:
                    pltpu.make_async_copy(
                        srcb.at[wid, pl.ds(0, SCH)], nx_s, semi).wait()
                    pltpu.make_async_copy(
                        dstb.at[wid, pl.ds(0, SCH)], nx_d, semi).wait()
                pltpu.async_copy(h_hbm.at[nx_s.at[b]], rows[b], sems[b])

    plsc.subcore_barrier()
    pltpu.sync_copy(agg_sh.at[pl.ds(r0, RPT)], out.at[cid, pl.ds(r0, RPT)])


_GS_FN = pl.kernel(
    _gs_body,
    out_type=jax.ShapeDtypeStruct((2, NP, D), jnp.float32),
    mesh=_SC_MESH,
    scratch_types=[
        pltpu.VMEM((SCH, B), jnp.int32),
        pltpu.VMEM((SCH, B), jnp.int32),
        pltpu.VMEM((SCH, B), jnp.int32),
        pltpu.VMEM((SCH, B), jnp.int32),
        pltpu.VMEM((B, D), jnp.float32),
        pltpu.VMEM((B, D), jnp.float32),
        pltpu.VMEM_SHARED((NP, D), jnp.float32),
        pltpu.SemaphoreType.DMA,
        pltpu.SemaphoreType.DMA,
        pltpu.SemaphoreType.DMA,
    ],
)


def _pre_body(dp_ref, x_ref, w_ref, ns_ref, nd_ref, h_ref):
    dp = dp_ref[...]
    dsrc = dp[0, :, 0:1] + dp[2, :, 0:1]
    ddst = dp[1, :, 0:1] + dp[3, :, 0:1]
    ns = jnp.where(dsrc > 0, lax.rsqrt(jnp.maximum(dsrc, 1.0)), 0.0)
    nd = jnp.where(ddst > 0, lax.rsqrt(jnp.maximum(ddst, 1.0)), 0.0)
    ns_bc = jnp.broadcast_to(ns, (BLK, D))
    nd_bc = jnp.broadcast_to(nd, (BLK, D))
    ns_ref[...] = ns_bc
    nd_ref[...] = nd_bc
    h_ref[...] = jnp.dot(x_ref[...] * ns_bc, w_ref[...],
                         preferred_element_type=jnp.float32)


def _pre_call(dp, xp, W1):
    return pl.pallas_call(
        _pre_body,
        grid=(NP // BLK,),
        in_specs=[
            pl.BlockSpec((4, BLK, D), lambda i: (0, i, 0)),
            pl.BlockSpec((BLK, D), lambda i: (i, 0)),
            pl.BlockSpec((D, D), lambda i: (0, 0)),
        ],
        out_specs=[pl.BlockSpec((BLK, D), lambda i: (i, 0))] * 3,
        out_shape=[jax.ShapeDtypeStruct((NP, D), jnp.float32)] * 3,
    )(dp, xp, W1)


def _mid_body(p_ref, nd_ref, ns_ref, b_ref, w_ref, h_ref):
    t = (p_ref[0] + p_ref[1]) * nd_ref[...] + b_ref[...]
    t = jnp.maximum(t, 0.0)
    h_ref[...] = jnp.dot(t * ns_ref[...], w_ref[...],
                         preferred_element_type=jnp.float32)


def _mid_call(p, nd, ns, b, Wn):
    return pl.pallas_call(
        _mid_body,
        grid=(NP // BLK,),
        in_specs=[
            pl.BlockSpec((2, BLK, D), lambda i: (0, i, 0)),
            pl.BlockSpec((BLK, D), lambda i: (i, 0)),
            pl.BlockSpec((BLK, D), lambda i: (i, 0)),
            pl.BlockSpec((1, D), lambda i: (0, 0)),
            pl.BlockSpec((D, D), lambda i: (0, 0)),
        ],
        out_specs=pl.BlockSpec((BLK, D), lambda i: (i, 0)),
        out_shape=jax.ShapeDtypeStruct((NP, D), jnp.float32),
    )(p, nd, ns, b, Wn)


def _fin_body(p_ref, nd_ref, b_ref, o_ref):
    o_ref[...] = (p_ref[0] + p_ref[1]) * nd_ref[...] + b_ref[...]


def _fin_call(p, nd, b):
    return pl.pallas_call(
        _fin_body,
        grid=(NP // BLK,),
        in_specs=[
            pl.BlockSpec((2, BLK, D), lambda i: (0, i, 0)),
            pl.BlockSpec((BLK, D), lambda i: (i, 0)),
            pl.BlockSpec((1, D), lambda i: (0, 0)),
        ],
        out_specs=pl.BlockSpec((BLK, D), lambda i: (i, 0)),
        out_shape=jax.ShapeDtypeStruct((NP, D), jnp.float32),
    )(p, nd, b)


def kernel(x, edge_index, W1, b1, W2, b2, W3, b3):
    src = edge_index[0]
    dst = edge_index[1]
    pad = NW * CH * B - E
    fill = DUMP + (jnp.arange(pad, dtype=jnp.int32) % (NP - DUMP))
    srcb = jnp.concatenate([src, fill]).reshape(NW, CH, B)
    dstb = jnp.concatenate([dst, fill]).reshape(NW, CH, B)
    xp = jnp.zeros((NP, D), jnp.float32).at[:N].set(x)
    zeros_rows = jnp.zeros((RPT, D), jnp.float32)
    ones_rows = jnp.ones((B, D), jnp.float32)

    degp = _DEG_FN(srcb, dstb, ones_rows, zeros_rows)
    dp = degp.reshape(4, NP, D)
    ns, nd, h = _pre_call(dp, xp, W1)

    p = _GS_FN(h, srcb, dstb, zeros_rows)
    h = _mid_call(p, nd, ns, b1.reshape(1, D), W2)
    p = _GS_FN(h, srcb, dstb, zeros_rows)
    h = _mid_call(p, nd, ns, b2.reshape(1, D), W3)
    p = _GS_FN(h, srcb, dstb, zeros_rows)
    out = _fin_call(p, nd, b3.reshape(1, D))

    return (out[:N], edge_index)


# speedup vs baseline: 9.8432x; 1.0149x over previous
import jax
import jax.numpy as jnp
from jax import lax
from jax.experimental import pallas as pl
from jax.experimental.pallas import tpu as pltpu
from jax.experimental.pallas import tpu_sc as plsc

N = 10000
D = 128
E = 320000
NP = 10240
DUMP = 10000
NW = 32
CH = 80
B = 128
RPT = NP // 16
BLK = 2048

_SC_MESH = plsc.VectorSubcoreMesh(
    core_axis_name="c", subcore_axis_name="s", num_cores=2, num_subcores=16
)


DEG_Q = 5


def _deg_body(srcb, dstb, ones_hbm, zeros_hbm, out, idx_v, ones_v, hist_sh,
              sem_s):
    cid = lax.axis_index("c")
    sid = lax.axis_index("s")
    wid = cid * 16 + sid
    pltpu.sync_copy(ones_hbm, ones_v)
    r0 = sid * RPT
    qch = CH // DEG_Q
    for which, blk in ((0, srcb), (1, dstb)):
        pltpu.sync_copy(zeros_hbm, hist_sh.at[pl.ds(r0, RPT)])
        plsc.subcore_barrier()
        for q in range(DEG_Q):
            pltpu.sync_copy(blk.at[wid, pl.ds(q * qch, qch)], idx_v)

            def fire(g, carry):
                pltpu.async_copy(ones_v, hist_sh.at[idx_v.at[g]], sem_s,
                                 add=True)
                return carry

            def drain(g, carry):
                pltpu.make_async_copy(ones_v, hist_sh.at[idx_v.at[0]],
                                      sem_s).wait()
                return carry

            lax.fori_loop(0, qch, fire, 0)
            lax.fori_loop(0, qch, drain, 0)
        plsc.subcore_barrier()
        pltpu.sync_copy(hist_sh.at[pl.ds(r0, RPT)],
                        out.at[cid, which, pl.ds(r0, RPT)])
        plsc.subcore_barrier()


_DEG_FN = pl.kernel(
    _deg_body,
    out_type=jax.ShapeDtypeStruct((2, 2, NP, D), jnp.float32),
    mesh=_SC_MESH,
    scratch_types=[
        pltpu.VMEM((CH // DEG_Q, B), jnp.int32),
        pltpu.VMEM((B, D), jnp.float32),
        pltpu.VMEM_SHARED((NP, D), jnp.float32),
        pltpu.SemaphoreType.DMA,
    ],
)


NSTG = 10
SCH = CH // NSTG


def _gs_body(h_hbm, srcb, dstb, zeros_hbm, out,
             src0, dst0, src1, dst1, rows0, rows1, agg_sh,
             sem0, sem1, semi):
    cid = lax.axis_index("c")
    sid = lax.axis_index("s")
    wid = cid * 16 + sid
    r0 = sid * RPT
    srcs, dsts = (src0, src1), (dst0, dst1)
    rows, sems = (rows0, rows1), (sem0, sem1)

    pltpu.sync_copy(srcb.at[wid, pl.ds(0, SCH)], src0)
    pltpu.sync_copy(dstb.at[wid, pl.ds(0, SCH)], dst0)
    pltpu.async_copy(h_hbm.at[src0.at[0]], rows0, sem0)
    pltpu.async_copy(h_hbm.at[src0.at[1]], rows1, sem1)
    pltpu.sync_copy(zeros_hbm, agg_sh.at[pl.ds(r0, RPT)])
    plsc.subcore_barrier()

    for s in range(NSTG):
        cs, cd = srcs[s % 2], dsts[s % 2]
        nx_s, nx_d = srcs[(s + 1) % 2], dsts[(s + 1) % 2]
        if s + 1 < NSTG:
            pltpu.async_copy(srcb.at[wid, pl.ds((s + 1) * SCH, SCH)],
                             nx_s, semi)
            pltpu.async_copy(dstb.at[wid, pl.ds((s + 1) * SCH, SCH)],
                             nx_d, semi)

        def pair(i, carry, cs=cs, cd=cd):
            g = 2 * i
            for b in range(2):
                j = g + b
                pltpu.make_async_copy(h_hbm.at[cs.at[j]],
                                      rows[b], sems[b]).wait()
                pltpu.sync_copy(rows[b], agg_sh.at[cd.at[j]], add=True)
                pltpu.async_copy(h_hbm.at[cs.at[j + 2]], rows[b], sems[b])
            return carry

        lax.fori_loop(0, SCH // 2 - 1, pair, 0)
        for b in range(2):
            j = SCH - 2 + b
            pltpu.make_async_copy(h_hbm.at[cs.at[j]], rows[b], sems[b]).wait()
            pltpu.sync_copy(rows[b], agg_sh.at[cd.at[j]], add=True)
            if s + 1 < NSTG:
                if b == 0:
                    pltpu.make_async_copy(
                        srcb.at[wid, pl.ds(0, SCH)], nx_s, semi).wait()
                    pltpu.make_async_copy(
                        dstb.at[wid, pl.ds(0, SCH)], nx_d, semi).wait()
                pltpu.async_copy(h_hbm.at[nx_s.at[b]], rows[b], sems[b])

    plsc.subcore_barrier()
    pltpu.sync_copy(agg_sh.at[pl.ds(r0, RPT)], out.at[cid, pl.ds(r0, RPT)])


_GS_FN = pl.kernel(
    _gs_body,
    out_type=jax.ShapeDtypeStruct((2, NP, D), jnp.float32),
    mesh=_SC_MESH,
    scratch_types=[
        pltpu.VMEM((SCH, B), jnp.int32),
        pltpu.VMEM((SCH, B), jnp.int32),
        pltpu.VMEM((SCH, B), jnp.int32),
        pltpu.VMEM((SCH, B), jnp.int32),
        pltpu.VMEM((B, D), jnp.float32),
        pltpu.VMEM((B, D), jnp.float32),
        pltpu.VMEM_SHARED((NP, D), jnp.float32),
        pltpu.SemaphoreType.DMA,
        pltpu.SemaphoreType.DMA,
        pltpu.SemaphoreType.DMA,
    ],
)


def _pre_body(dp_ref, x_ref, w_ref, ns_ref, nd_ref, h_ref):
    dp = dp_ref[...]
    dsrc = dp[0, :, 0:1] + dp[2, :, 0:1]
    ddst = dp[1, :, 0:1] + dp[3, :, 0:1]
    ns = jnp.where(dsrc > 0, lax.rsqrt(jnp.maximum(dsrc, 1.0)), 0.0)
    nd = jnp.where(ddst > 0, lax.rsqrt(jnp.maximum(ddst, 1.0)), 0.0)
    ns_bc = jnp.broadcast_to(ns, (BLK, D))
    nd_bc = jnp.broadcast_to(nd, (BLK, D))
    ns_ref[...] = ns_bc
    nd_ref[...] = nd_bc
    h_ref[...] = jnp.dot(x_ref[...] * ns_bc, w_ref[...],
                         preferred_element_type=jnp.float32)


def _pre_call(dp, xp, W1):
    return pl.pallas_call(
        _pre_body,
        grid=(NP // BLK,),
        in_specs=[
            pl.BlockSpec((4, BLK, D), lambda i: (0, i, 0)),
            pl.BlockSpec((BLK, D), lambda i: (i, 0)),
            pl.BlockSpec((D, D), lambda i: (0, 0)),
        ],
        out_specs=[pl.BlockSpec((BLK, D), lambda i: (i, 0))] * 3,
        out_shape=[jax.ShapeDtypeStruct((NP, D), jnp.float32)] * 3,
    )(dp, xp, W1)


def _mid_body(p_ref, nd_ref, ns_ref, b_ref, w_ref, h_ref):
    t = (p_ref[0] + p_ref[1]) * nd_ref[...] + b_ref[...]
    t = jnp.maximum(t, 0.0)
    h_ref[...] = jnp.dot(t * ns_ref[...], w_ref[...],
                         preferred_element_type=jnp.float32)


def _mid_call(p, nd, ns, b, Wn):
    return pl.pallas_call(
        _mid_body,
        grid=(NP // BLK,),
        in_specs=[
            pl.BlockSpec((2, BLK, D), lambda i: (0, i, 0)),
            pl.BlockSpec((BLK, D), lambda i: (i, 0)),
            pl.BlockSpec((BLK, D), lambda i: (i, 0)),
            pl.BlockSpec((1, D), lambda i: (0, 0)),
            pl.BlockSpec((D, D), lambda i: (0, 0)),
        ],
        out_specs=pl.BlockSpec((BLK, D), lambda i: (i, 0)),
        out_shape=jax.ShapeDtypeStruct((NP, D), jnp.float32),
    )(p, nd, ns, b, Wn)


def _fin_body(p_ref, nd_ref, b_ref, o_ref):
    o_ref[...] = (p_ref[0] + p_ref[1]) * nd_ref[...] + b_ref[...]


def _fin_call(p, nd, b):
    return pl.pallas_call(
        _fin_body,
        grid=(NP // BLK,),
        in_specs=[
            pl.BlockSpec((2, BLK, D), lambda i: (0, i, 0)),
            pl.BlockSpec((BLK, D), lambda i: (i, 0)),
            pl.BlockSpec((1, D), lambda i: (0, 0)),
        ],
        out_specs=pl.BlockSpec((BLK, D), lambda i: (i, 0)),
        out_shape=jax.ShapeDtypeStruct((NP, D), jnp.float32),
    )(p, nd, b)


def kernel(x, edge_index, W1, b1, W2, b2, W3, b3):
    src = edge_index[0]
    dst = edge_index[1]
    pad = NW * CH * B - E
    fill = DUMP + (jnp.arange(pad, dtype=jnp.int32) % (NP - DUMP))
    srcb = jnp.concatenate([src, fill]).reshape(NW, CH, B)
    dstb = jnp.concatenate([dst, fill]).reshape(NW, CH, B)
    xp = jnp.zeros((NP, D), jnp.float32).at[:N].set(x)
    zeros_rows = jnp.zeros((RPT, D), jnp.float32)
    ones_rows = jnp.ones((B, D), jnp.float32)

    degp = _DEG_FN(srcb, dstb, ones_rows, zeros_rows)
    dp = degp.reshape(4, NP, D)
    ns, nd, h = _pre_call(dp, xp, W1)

    p = _GS_FN(h, srcb, dstb, zeros_rows)
    h = _mid_call(p, nd, ns, b1.reshape(1, D), W2)
    p = _GS_FN(h, srcb, dstb, zeros_rows)
    h = _mid_call(p, nd, ns, b2.reshape(1, D), W3)
    p = _GS_FN(h, srcb, dstb, zeros_rows)
    out = _fin_call(p, nd, b3.reshape(1, D))

    return (out[:N], edge_index)
